# baseline (speedup 1.0000x reference)
import jax
import jax.numpy as jnp
from jax import lax
from jax.experimental import pallas as pl
from jax.experimental.pallas import tpu as pltpu
from jax.experimental.pallas import tpu_sc as plsc

_B = 16384
_D = 128
_NC = 2
_NS = 16
_NW = _NC * _NS
_BPW = _B // _NW
_CH = 128
_NCHUNK = _BPW // _CH


def _gmf_body(users_hbm, items_hbm, utab_hbm, itab_hbm, w_hbm, binit_hbm,
              out_hbm, uidx_v, iidx_v, urows_v, irows_v, w_v, binit_v,
              tmp_v, pos_v, outc_v, sem):
    cid = lax.axis_index("c")
    sid = lax.axis_index("s")
    wid = sid * _NC + cid
    base = wid * _BPW

    pltpu.sync_copy(w_hbm, w_v)
    pltpu.sync_copy(binit_hbm, binit_v)
    b_init = binit_v[...]
    w_regs = [w_v[pl.ds(j * 16, 16)] for j in range(8)]
    lane = lax.iota(jnp.int32, 16)
    zeros = jnp.zeros((16,), jnp.float32)

    for chunk in range(_NCHUNK):
        cbase = base + chunk * _CH
        pltpu.sync_copy(users_hbm.at[pl.ds(cbase, _CH)], uidx_v)
        pltpu.sync_copy(items_hbm.at[pl.ds(cbase, _CH)], iidx_v)
        cp_u = pltpu.async_copy(utab_hbm.at[uidx_v], urows_v, sem)
        cp_i = pltpu.async_copy(itab_hbm.at[iidx_v], irows_v, sem)
        cp_u.wait()
        cp_i.wait()

        def group_body(g, carry):
            rbase = g * 16
            for rr in range(16):
                r = rbase + rr
                acc = b_init
                for j in range(8):
                    acc = acc + (urows_v[r, pl.ds(j * 16, 16)]
                                 * irows_v[r, pl.ds(j * 16, 16)]
                                 * w_regs[j])
                t = rr * 32
                tmp_v[pl.ds(t, 16)] = acc
                s1 = acc + tmp_v[pl.ds(t + 8, 16)]
                tmp_v[pl.ds(t, 16)] = s1
                s2 = s1 + tmp_v[pl.ds(t + 4, 16)]
                tmp_v[pl.ds(t, 16)] = s2
                s3 = s2 + tmp_v[pl.ds(t + 2, 16)]
                tmp_v[pl.ds(t, 16)] = s3
                s4 = s3 + tmp_v[pl.ds(t + 1, 16)]
                pos_v[pl.ds(rr, 16)] = s4
            outc_v[pl.ds(rbase, 16)] = pos_v[pl.ds(0, 16)]
            return carry

        lax.fori_loop(0, _CH // 16, group_body, 0)
        pltpu.sync_copy(outc_v, out_hbm.at[pl.ds(cbase, _CH)])


def kernel(users, items, user_table, item_table, W_beta, b_beta):
    users_i = users.astype(jnp.int32)
    items_i = items.astype(jnp.int32)
    w = W_beta.reshape(_D)
    binit = jnp.pad(b_beta.reshape(1), (0, 15))

    mesh = plsc.VectorSubcoreMesh(core_axis_name="c", subcore_axis_name="s")
    f = pl.kernel(
        _gmf_body,
        mesh=mesh,
        out_type=jax.ShapeDtypeStruct((_B,), jnp.float32),
        scratch_types=[
            pltpu.VMEM((_CH,), jnp.int32),
            pltpu.VMEM((_CH,), jnp.int32),
            pltpu.VMEM((_CH, _D), jnp.float32),
            pltpu.VMEM((_CH, _D), jnp.float32),
            pltpu.VMEM((_D,), jnp.float32),
            pltpu.VMEM((16,), jnp.float32),
            pltpu.VMEM((512,), jnp.float32),
            pltpu.VMEM((32,), jnp.float32),
            pltpu.VMEM((_CH,), jnp.float32),
            pltpu.SemaphoreType.DMA,
        ],
    )
    out = f(users_i, items_i, user_table, item_table, w, binit)
    return out.reshape(_B, 1)


# speedup vs baseline: 1.1364x; 1.1364x over previous
import jax
import jax.numpy as jnp
from jax import lax
from jax.experimental import pallas as pl
from jax.experimental.pallas import tpu as pltpu
from jax.experimental.pallas import tpu_sc as plsc

_B = 16384
_D = 128
_NC = 2
_NS = 16
_NW = _NC * _NS
_BPW = _B // _NW
_CH = 128
_NCHUNK = _BPW // _CH


def _gmf_body(users_hbm, items_hbm, utab_hbm, itab_hbm, w_hbm, binit_hbm,
              out_hbm, uidx0, uidx1, iidx0, iidx1, urows0, urows1,
              irows0, irows1, w_v, binit_v, tmp_v, pos_v, outc_v,
              sem0, sem1):
    cid = lax.axis_index("c")
    sid = lax.axis_index("s")
    wid = sid * _NC + cid
    base = wid * _BPW

    pltpu.sync_copy(w_hbm, w_v)
    pltpu.sync_copy(binit_hbm, binit_v)
    b_init = binit_v[...]
    w_regs = [w_v[pl.ds(j * 16, 16)] for j in range(8)]

    ubufs = (urows0, urows1)
    ibufs = (irows0, irows1)
    uidxs = (uidx0, uidx1)
    iidxs = (iidx0, iidx1)
    sems = (sem0, sem1)
    pending = [None, None]

    def start(c):
        k = c % 2
        cbase = base + c * _CH
        pltpu.sync_copy(users_hbm.at[pl.ds(cbase, _CH)], uidxs[k])
        pltpu.sync_copy(items_hbm.at[pl.ds(cbase, _CH)], iidxs[k])
        cu = pltpu.async_copy(utab_hbm.at[uidxs[k]], ubufs[k], sems[k])
        ci = pltpu.async_copy(itab_hbm.at[iidxs[k]], ibufs[k], sems[k])
        pending[k] = (cu, ci)

    start(0)
    for c in range(_NCHUNK):
        if c + 1 < _NCHUNK:
            start(c + 1)
        k = c % 2
        cu, ci = pending[k]
        cu.wait()
        ci.wait()
        urows_v = ubufs[k]
        irows_v = ibufs[k]
        cbase = base + c * _CH

        def group_body(g, carry):
            rbase = g * 16
            for rr in range(16):
                r = rbase + rr
                acc = b_init
                for j in range(8):
                    acc = acc + (urows_v[r, pl.ds(j * 16, 16)]
                                 * irows_v[r, pl.ds(j * 16, 16)]
                                 * w_regs[j])
                t = rr * 32
                tmp_v[pl.ds(t, 16)] = acc
                s1 = acc + tmp_v[pl.ds(t + 8, 16)]
                tmp_v[pl.ds(t, 16)] = s1
                s2 = s1 + tmp_v[pl.ds(t + 4, 16)]
                tmp_v[pl.ds(t, 16)] = s2
                s3 = s2 + tmp_v[pl.ds(t + 2, 16)]
                tmp_v[pl.ds(t, 16)] = s3
                s4 = s3 + tmp_v[pl.ds(t + 1, 16)]
                pos_v[pl.ds(rr, 16)] = s4
            outc_v[pl.ds(rbase, 16)] = pos_v[pl.ds(0, 16)]
            return carry

        lax.fori_loop(0, _CH // 16, group_body, 0)
        pltpu.sync_copy(outc_v, out_hbm.at[pl.ds(cbase, _CH)])


def kernel(users, items, user_table, item_table, W_beta, b_beta):
    users_i = users.astype(jnp.int32)
    items_i = items.astype(jnp.int32)
    w = W_beta.reshape(_D)
    binit = jnp.pad(b_beta.reshape(1), (0, 15))

    mesh = plsc.VectorSubcoreMesh(core_axis_name="c", subcore_axis_name="s")
    f = pl.kernel(
        _gmf_body,
        mesh=mesh,
        out_type=jax.ShapeDtypeStruct((_B,), jnp.float32),
        scratch_types=[
            pltpu.VMEM((_CH,), jnp.int32),
            pltpu.VMEM((_CH,), jnp.int32),
            pltpu.VMEM((_CH,), jnp.int32),
            pltpu.VMEM((_CH,), jnp.int32),
            pltpu.VMEM((_CH, _D), jnp.float32),
            pltpu.VMEM((_CH, _D), jnp.float32),
            pltpu.VMEM((_CH, _D), jnp.float32),
            pltpu.VMEM((_CH, _D), jnp.float32),
            pltpu.VMEM((_D,), jnp.float32),
            pltpu.VMEM((16,), jnp.float32),
            pltpu.VMEM((512,), jnp.float32),
            pltpu.VMEM((32,), jnp.float32),
            pltpu.VMEM((_CH,), jnp.float32),
            pltpu.SemaphoreType.DMA,
            pltpu.SemaphoreType.DMA,
        ],
    )
    out = f(users_i, items_i, user_table, item_table, w, binit)
    return out.reshape(_B, 1)
